# Initial kernel scaffold; baseline (speedup 1.0000x reference)
import jax
import jax.numpy as jnp
from jax.experimental import pallas as pl


def kernel(x):
    raise NotImplementedError("write your pallas kernel here")


# speedup vs baseline: 8.7330x; 8.7330x over previous
import functools

import jax
import jax.numpy as jnp
from jax.experimental import pallas as pl

_K = 16
_NUM_SAMPLES = 100
_SIGMA = 0.05
_B = 16
_D = 2048


def _ptopk_kernel(x_ref, noise_ref, out_ref):
    x_row = x_ref[0, 0, :]
    work = x_row[None, :] + noise_ref[0] * _SIGMA
    iota = jax.lax.broadcasted_iota(jnp.int32, (_NUM_SAMPLES, _D), 1)
    inv_n = jnp.float32(1.0 / _NUM_SAMPLES)
    for k in range(_K):
        v = jnp.max(work, axis=1, keepdims=True)
        is_max = work == v
        idx = jnp.min(jnp.where(is_max, iota, _D), axis=1, keepdims=True)
        sel = iota == idx
        out_ref[0, k, :] = jnp.sum(sel.astype(jnp.float32), axis=0) * inv_n
        work = jnp.where(sel, -jnp.inf, work)


@functools.partial(jax.jit, static_argnames=())
def kernel(x):
    b, d = x.shape
    noise = jax.random.normal(
        jax.random.key(1), (b, _NUM_SAMPLES, d), dtype=jnp.float32)
    return pl.pallas_call(
        _ptopk_kernel,
        grid=(b,),
        in_specs=[
            pl.BlockSpec((1, 1, d), lambda i: (i, 0, 0)),
            pl.BlockSpec((1, _NUM_SAMPLES, d), lambda i: (i, 0, 0)),
        ],
        out_specs=pl.BlockSpec((1, _K, d), lambda i: (i, 0, 0)),
        out_shape=jax.ShapeDtypeStruct((b, _K, d), jnp.float32),
    )(x.reshape(b, 1, d), noise)
